# Initial kernel scaffold; baseline (speedup 1.0000x reference)
import jax
import jax.numpy as jnp
from jax.experimental import pallas as pl


def kernel(words, table):
    raise NotImplementedError("write your pallas kernel here")


# speedup vs baseline: 4.9413x; 4.9413x over previous
import functools

import jax
import jax.numpy as jnp
from jax import lax
from jax.experimental import pallas as pl
from jax.experimental.pallas import tpu as pltpu
from jax.experimental.pallas import tpu_sc as plsc

_INFO = plsc.get_sparse_core_info()
_NC = _INFO.num_cores
_NS = _INFO.num_subcores
_NW = _NC * _NS

_CHUNK = 2048


@functools.partial(jax.jit, static_argnums=(2, 3))
def _gather_rows(words_flat, table, n, d):
    per_w = n // _NW
    steps = per_w // _CHUNK
    mesh = plsc.VectorSubcoreMesh(core_axis_name="c", subcore_axis_name="s")

    @functools.partial(
        pl.kernel,
        out_type=jax.ShapeDtypeStruct((n, d), jnp.float32),
        mesh=mesh,
        scratch_types=[
            pltpu.VMEM((_CHUNK,), jnp.int32),
            pltpu.VMEM((_CHUNK, d), jnp.float32),
            pltpu.SemaphoreType.DMA,
        ],
        compiler_params=pltpu.CompilerParams(use_tc_tiling_on_sc=False),
    )
    def k(words_hbm, table_hbm, out_hbm, idx_v, rows_v, sem):
        wid = lax.axis_index("s") * _NC + lax.axis_index("c")
        base = wid * per_w

        def body(i, carry):
            off = base + i * _CHUNK
            pltpu.sync_copy(words_hbm.at[pl.ds(off, _CHUNK)], idx_v)
            pltpu.async_copy(table_hbm.at[idx_v], rows_v, sem).wait()
            pltpu.sync_copy(rows_v, out_hbm.at[pl.ds(off, _CHUNK)])
            return carry

        lax.fori_loop(0, steps, body, 0)

    return k(words_flat, table)


def kernel(words, table):
    b, h = words.shape
    v, d = table.shape
    n = b * h
    flat = words.reshape(n).astype(jnp.int32)
    out = _gather_rows(flat, table, n, d)
    return out.reshape(b, h, d)


# speedup vs baseline: 5.0093x; 1.0138x over previous
import functools

import jax
import jax.numpy as jnp
from jax import lax
from jax.experimental import pallas as pl
from jax.experimental.pallas import tpu as pltpu
from jax.experimental.pallas import tpu_sc as plsc

_INFO = plsc.get_sparse_core_info()
_NC = _INFO.num_cores
_NS = _INFO.num_subcores
_NW = _NC * _NS

_CHUNK = 1600
_NBUF = 2


@functools.partial(jax.jit, static_argnums=(2, 3))
def _gather_rows(words_flat, table, n, d):
    per_w = n // _NW
    steps = per_w // _CHUNK
    groups = steps // _NBUF
    mesh = plsc.VectorSubcoreMesh(core_axis_name="c", subcore_axis_name="s")

    @functools.partial(
        pl.kernel,
        out_type=jax.ShapeDtypeStruct((n, d), jnp.float32),
        mesh=mesh,
        scratch_types=[
            pltpu.VMEM((_NBUF, _CHUNK), jnp.int32),
            pltpu.VMEM((_NBUF, _CHUNK, d), jnp.float32),
            pltpu.SemaphoreType.DMA((_NBUF,)),
            pltpu.SemaphoreType.DMA((_NBUF,)),
            pltpu.SemaphoreType.DMA((_NBUF,)),
        ],
        compiler_params=pltpu.CompilerParams(use_tc_tiling_on_sc=False),
    )
    def k(words_hbm, table_hbm, out_hbm, idx_v, rows_v, sem_i, sem_g, sem_o):
        wid = lax.axis_index("s") * _NC + lax.axis_index("c")
        base = wid * per_w

        def start_idx(g, b):
            pltpu.async_copy(
                words_hbm.at[pl.ds(base + g * _CHUNK, _CHUNK)],
                idx_v.at[b], sem_i.at[b])

        def wait_idx(b):
            pltpu.make_async_copy(
                words_hbm.at[pl.ds(base, _CHUNK)],
                idx_v.at[b], sem_i.at[b]).wait()

        def start_gather(b):
            pltpu.async_copy(
                table_hbm.at[idx_v.at[b]], rows_v.at[b], sem_g.at[b])

        def wait_gather(b):
            pltpu.make_async_copy(
                table_hbm.at[idx_v.at[b]], rows_v.at[b], sem_g.at[b]).wait()

        def start_out(g, b):
            pltpu.async_copy(
                rows_v.at[b],
                out_hbm.at[pl.ds(base + g * _CHUNK, _CHUNK)], sem_o.at[b])

        def wait_out(b):
            pltpu.make_async_copy(
                rows_v.at[b],
                out_hbm.at[pl.ds(base, _CHUNK)], sem_o.at[b]).wait()

        for b in range(_NBUF):
            start_idx(b, b)

        def body(i, carry):
            g0 = i * _NBUF
            for b in range(_NBUF):
                wait_idx(b)

                @pl.when(i > 0)
                def _():
                    wait_out(b)

                start_gather(b)
            for b in range(_NBUF):
                wait_gather(b)

                @pl.when(i + 1 < groups)
                def _():
                    start_idx(g0 + _NBUF + b, b)

                start_out(g0 + b, b)
            return carry

        lax.fori_loop(0, groups, body, 0)

        for b in range(_NBUF):
            wait_out(b)

    return k(words_flat, table)


def kernel(words, table):
    b, h = words.shape
    v, d = table.shape
    n = b * h
    flat = words.reshape(n).astype(jnp.int32)
    out = _gather_rows(flat, table, n, d)
    return out.reshape(b, h, d)
